# baseline (speedup 1.0000x reference)
import functools

import jax
import jax.numpy as jnp
from jax import lax
from jax.experimental import pallas as pl
from jax.experimental.pallas import tpu as pltpu
from jax.experimental.pallas import tpu_sc as plsc

DIM = 768
E = 64
DFF = 768
T = 2048
CAP = 40
SLOTS = E * CAP
NC = 2
NS = 16
NW = NC * NS
SPW = SLOTS // NW
TPW = T // NW
SLOTS_PAD = 2576


def _route_body(tok_ref, gw_ref, slot_ref, wt_ref, laux_ref):
    x = tok_ref[...]
    g = gw_ref[...]
    logits = lax.dot_general(g, x, (((1,), (1,)), ((), ())),
                             preferred_element_type=jnp.float32)
    m = jnp.max(logits, axis=0, keepdims=True)
    ex = jnp.exp(logits - m)
    probs = ex / jnp.sum(ex, axis=0, keepdims=True)
    pmax = jnp.max(probs, axis=0, keepdims=True)
    iota_e = lax.broadcasted_iota(jnp.int32, (E, T), 0)
    cand = jnp.where(probs == pmax, iota_e, jnp.int32(E))
    top1 = jnp.min(cand, axis=0, keepdims=True)
    mask = (iota_e == top1).astype(jnp.float32)
    c = mask
    shift = 1
    while shift < T:
        c = c + jnp.concatenate(
            [jnp.zeros((E, shift), jnp.float32), c[:, :T - shift]], axis=1)
        shift *= 2
    rank = jnp.sum((c - 1.0) * mask, axis=0)
    kept = rank < float(CAP)
    wsel = jnp.sum(probs * mask, axis=0)
    slot = jnp.where(kept,
                     jnp.min(cand, axis=0) * CAP + rank.astype(jnp.int32),
                     jnp.int32(SLOTS))
    me = jnp.sum(probs, axis=1, keepdims=True) / float(T)
    ce = jnp.sum(mask, axis=1, keepdims=True) / float(T)
    laux_ref[...] = float(E) * jnp.sum(me * ce, axis=0, keepdims=True)
    slot_ref[...] = slot
    wt_ref[...] = wsel


def _route(tokens, gate_weight):
    return pl.pallas_call(
        _route_body,
        out_shape=(
            jax.ShapeDtypeStruct((T,), jnp.int32),
            jax.ShapeDtypeStruct((T,), jnp.float32),
            jax.ShapeDtypeStruct((1, 1), jnp.float32),
        ),
    )(tokens, gate_weight)


_UNROLL = 8
_HALF = SPW // 2


def _dispatch_body(tok_hbm, slot_hbm, wt_hbm, disp_hbm, swt_hbm,
                   slot_v, wt_v, s2t_v, sw_v, rows0_v, rows1_v,
                   sem0, sem1, sem2):
    wid = lax.axis_index("s") * NC + lax.axis_index("c")
    base = wid * SPW
    with jax.named_scope("disp_load_in"):
        lw = pltpu.async_copy(wt_hbm, wt_v, sem2)
        pltpu.sync_copy(slot_hbm, slot_v)

    for i in range(SPW // 16):
        s2t_v[pl.ds(base + i * 16, 16)] = (
            (lax.iota(jnp.int32, 16) + (base + i * 16)) & (T - 1))

    def scatter(i, _):
        for j in range(_UNROLL):
            k = i * _UNROLL + j
            idx = slot_v[pl.ds(k * 16, 16)]
            plsc.store_scatter(s2t_v, [idx],
                               lax.iota(jnp.int32, 16) + k * 16)
        return 0

    with jax.named_scope("disp_scatter"):
        lax.fori_loop(0, T // 16 // _UNROLL, scatter, 0)

    g0 = pltpu.async_copy(tok_hbm.at[s2t_v.at[pl.ds(base, _HALF)]],
                          rows0_v, sem0)
    g1 = pltpu.async_copy(tok_hbm.at[s2t_v.at[pl.ds(base + _HALF, _HALF)]],
                          rows1_v, sem1)
    with jax.named_scope("disp_sw"):
        lw.wait()
        for i in range(SPW // 16):
            idx = s2t_v[pl.ds(base + i * 16, 16)]
            sw_v[pl.ds(i * 16, 16)] = plsc.load_gather(wt_v, [idx])
        pltpu.sync_copy(sw_v, swt_hbm.at[pl.ds(base, SPW)])
    with jax.named_scope("disp_rows"):
        g0.wait()
        w0 = pltpu.async_copy(rows0_v, disp_hbm.at[pl.ds(base, _HALF)], sem0)
        g1.wait()
        w1 = pltpu.async_copy(rows1_v,
                              disp_hbm.at[pl.ds(base + _HALF, _HALF)], sem1)
        w0.wait()
        w1.wait()


@functools.lru_cache(maxsize=None)
def _dispatch_kernel():
    return pl.kernel(
        _dispatch_body,
        out_type=(
            jax.ShapeDtypeStruct((SLOTS, DIM), jnp.float32),
            jax.ShapeDtypeStruct((SLOTS,), jnp.float32),
        ),
        mesh=plsc.VectorSubcoreMesh(core_axis_name="c", subcore_axis_name="s",
                                    num_cores=NC, num_subcores=NS),
        scratch_types=(
            pltpu.VMEM((T,), jnp.int32),
            pltpu.VMEM((T,), jnp.float32),
            pltpu.VMEM((SLOTS_PAD,), jnp.int32),
            pltpu.VMEM((SPW,), jnp.float32),
            pltpu.VMEM((_HALF, DIM), jnp.float32),
            pltpu.VMEM((_HALF, DIM), jnp.float32),
            pltpu.SemaphoreType.DMA,
            pltpu.SemaphoreType.DMA,
            pltpu.SemaphoreType.DMA,
        ),
        compiler_params=pltpu.CompilerParams(needs_layout_passes=False),
    )


_EB = 4
_NSTEP = E // _EB + 1


def _ffn_body(disp_ref, w1_ref, b1_ref, w2_ref, b2_ref, swt_ref, out_ref):
    g = pl.program_id(0)

    @pl.when(g < E // _EB)
    def _():
        for j in range(_EB):
            x = disp_ref[pl.ds(j * CAP, CAP), :]
            h = lax.dot_general(x, w1_ref[j], (((1,), (0,)), ((), ())),
                                preferred_element_type=jnp.float32)
            h = jnp.maximum(h + b1_ref[pl.ds(g * _EB + j, 1), :], 0.0)
            y = lax.dot_general(h, w2_ref[j], (((1,), (0,)), ((), ())),
                                preferred_element_type=jnp.float32)
            y = y + b2_ref[pl.ds(g * _EB + j, 1), :]
            out_ref[pl.ds(j * CAP, CAP), :] = y * swt_ref[pl.ds(j * CAP, CAP), :]

    @pl.when(g == E // _EB)
    def _():
        out_ref[...] = jnp.zeros((_EB * CAP, DIM), jnp.float32)


def _ffn(disp, w1, b1, w2, b2, slot_weight):
    clamp3 = lambda g: (jnp.minimum(g, E // _EB - 1), 0, 0)
    clamp2 = lambda g: (jnp.minimum(g, E // _EB - 1), 0)
    return pl.pallas_call(
        _ffn_body,
        grid=(_NSTEP,),
        in_specs=[
            pl.BlockSpec((_EB * CAP, DIM), clamp2),
            pl.BlockSpec((_EB, DIM, DFF), clamp3),
            pl.BlockSpec((E, DFF), lambda g: (0, 0)),
            pl.BlockSpec((_EB, DFF, DIM), clamp3),
            pl.BlockSpec((E, DIM), lambda g: (0, 0)),
            pl.BlockSpec((_EB * CAP, 1), clamp2),
        ],
        out_specs=pl.BlockSpec((_EB * CAP, DIM), lambda g: (g, 0)),
        out_shape=jax.ShapeDtypeStruct((SLOTS + _EB * CAP, DIM), jnp.float32),
    )(disp, w1, b1, w2, b2, slot_weight)


_NCHUNK = 4
_CSZ = TPW // _NCHUNK


def _combine_body(eo_hbm, slot_hbm, out_hbm, idx_v,
                  rows0_v, rows1_v, rows2_v, rows3_v,
                  sem0, sem1, sem2, sem3):
    wid = lax.axis_index("s") * NC + lax.axis_index("c")
    base = wid * TPW
    pltpu.sync_copy(slot_hbm.at[pl.ds(base, TPW)], idx_v)
    rows = (rows0_v, rows1_v, rows2_v, rows3_v)
    sems = (sem0, sem1, sem2, sem3)
    gs = [pltpu.async_copy(eo_hbm.at[idx_v.at[pl.ds(i * _CSZ, _CSZ)]],
                           rows[i], sems[i])
          for i in range(_NCHUNK)]
    ws = []
    for i in range(_NCHUNK):
        gs[i].wait()
        ws.append(pltpu.async_copy(
            rows[i], out_hbm.at[pl.ds(base + i * _CSZ, _CSZ)], sems[i]))
    for w in ws:
        w.wait()


@functools.lru_cache(maxsize=None)
def _combine_kernel():
    return pl.kernel(
        _combine_body,
        out_type=jax.ShapeDtypeStruct((T, DIM), jnp.float32),
        mesh=plsc.VectorSubcoreMesh(core_axis_name="c", subcore_axis_name="s",
                                    num_cores=NC, num_subcores=NS),
        scratch_types=(
            pltpu.VMEM((TPW,), jnp.int32),
            pltpu.VMEM((_CSZ, DIM), jnp.float32),
            pltpu.VMEM((_CSZ, DIM), jnp.float32),
            pltpu.VMEM((_CSZ, DIM), jnp.float32),
            pltpu.VMEM((_CSZ, DIM), jnp.float32),
            pltpu.SemaphoreType.DMA,
            pltpu.SemaphoreType.DMA,
            pltpu.SemaphoreType.DMA,
            pltpu.SemaphoreType.DMA,
        ),
        compiler_params=pltpu.CompilerParams(needs_layout_passes=False),
    )


def kernel(inputs, gate_weight, w1, b1, w2, b2):
    tokens = inputs.reshape(T, DIM)
    token_slot, token_wt, laux = _route(tokens, gate_weight)
    disp, slot_weight = _dispatch_kernel()(tokens, token_slot, token_wt)
    eo = _ffn(disp, w1, b1, w2, b2, slot_weight.reshape(SLOTS, 1))
    ans = _combine_kernel()(eo, token_slot)
    return ans.reshape(inputs.shape), laux.reshape(())


# speedup vs baseline: 1.0192x; 1.0192x over previous
import functools

import jax
import jax.numpy as jnp
from jax import lax
from jax.experimental import pallas as pl
from jax.experimental.pallas import tpu as pltpu
from jax.experimental.pallas import tpu_sc as plsc

DIM = 768
E = 64
DFF = 768
T = 2048
CAP = 40
SLOTS = E * CAP
NC = 2
NS = 16
NW = NC * NS
SPW = SLOTS // NW
TPW = T // NW
SLOTS_PAD = 2576


def _route_body(tok_ref, gw_ref, slot_ref, wt_ref, laux_ref):
    x = tok_ref[...]
    g = gw_ref[...]
    logits = lax.dot_general(g, x, (((1,), (1,)), ((), ())),
                             preferred_element_type=jnp.float32)
    m = jnp.max(logits, axis=0, keepdims=True)
    ex = jnp.exp(logits - m)
    probs = ex / jnp.sum(ex, axis=0, keepdims=True)
    pmax = jnp.max(probs, axis=0, keepdims=True)
    iota_e = lax.broadcasted_iota(jnp.int32, (E, T), 0)
    cand = jnp.where(probs == pmax, iota_e, jnp.int32(E))
    top1 = jnp.min(cand, axis=0, keepdims=True)
    mask = (iota_e == top1).astype(jnp.float32)
    c = mask
    shift = 1
    while shift < T:
        c = c + jnp.concatenate(
            [jnp.zeros((E, shift), jnp.float32), c[:, :T - shift]], axis=1)
        shift *= 2
    rank = jnp.sum((c - 1.0) * mask, axis=0)
    kept = rank < float(CAP)
    wsel = jnp.sum(probs * mask, axis=0)
    slot = jnp.where(kept,
                     jnp.min(cand, axis=0) * CAP + rank.astype(jnp.int32),
                     jnp.int32(SLOTS))
    me = jnp.sum(probs, axis=1, keepdims=True) / float(T)
    ce = jnp.sum(mask, axis=1, keepdims=True) / float(T)
    laux_ref[...] = float(E) * jnp.sum(me * ce, axis=0, keepdims=True)
    slot_ref[...] = slot
    wt_ref[...] = wsel


def _route(tokens, gate_weight):
    return pl.pallas_call(
        _route_body,
        out_shape=(
            jax.ShapeDtypeStruct((T,), jnp.int32),
            jax.ShapeDtypeStruct((T,), jnp.float32),
            jax.ShapeDtypeStruct((1, 1), jnp.float32),
        ),
    )(tokens, gate_weight)


_UNROLL = 8
_HALF = SPW // 2


def _dispatch_body(tok_hbm, slot_hbm, wt_hbm, disp_hbm, swt_hbm,
                   slot_v, wt_v, s2t_v, sw_v, rows0_v, rows1_v,
                   sem0, sem1, sem2):
    wid = lax.axis_index("s") * NC + lax.axis_index("c")
    base = wid * SPW
    with jax.named_scope("disp_load_in"):
        lw = pltpu.async_copy(wt_hbm, wt_v, sem2)
        pltpu.sync_copy(slot_hbm, slot_v)

    for i in range(SPW // 16):
        s2t_v[pl.ds(base + i * 16, 16)] = (
            (lax.iota(jnp.int32, 16) + (base + i * 16)) & (T - 1))

    def scatter(i, _):
        for j in range(_UNROLL):
            k = i * _UNROLL + j
            idx = slot_v[pl.ds(k * 16, 16)]
            plsc.store_scatter(s2t_v, [idx],
                               lax.iota(jnp.int32, 16) + k * 16)
        return 0

    with jax.named_scope("disp_scatter"):
        lax.fori_loop(0, T // 16 // _UNROLL, scatter, 0)

    g0 = pltpu.async_copy(tok_hbm.at[s2t_v.at[pl.ds(base, _HALF)]],
                          rows0_v, sem0)
    g1 = pltpu.async_copy(tok_hbm.at[s2t_v.at[pl.ds(base + _HALF, _HALF)]],
                          rows1_v, sem1)
    with jax.named_scope("disp_sw"):
        lw.wait()
        for i in range(SPW // 16):
            idx = s2t_v[pl.ds(base + i * 16, 16)]
            sw_v[pl.ds(i * 16, 16)] = plsc.load_gather(wt_v, [idx])
        pltpu.sync_copy(sw_v, swt_hbm.at[pl.ds(base, SPW)])
    with jax.named_scope("disp_rows"):
        g0.wait()
        w0 = pltpu.async_copy(rows0_v, disp_hbm.at[pl.ds(base, _HALF)], sem0)
        g1.wait()
        w1 = pltpu.async_copy(rows1_v,
                              disp_hbm.at[pl.ds(base + _HALF, _HALF)], sem1)
        w0.wait()
        w1.wait()


@functools.lru_cache(maxsize=None)
def _dispatch_kernel():
    return pl.kernel(
        _dispatch_body,
        out_type=(
            jax.ShapeDtypeStruct((SLOTS, DIM), jnp.float32),
            jax.ShapeDtypeStruct((SLOTS,), jnp.float32),
        ),
        mesh=plsc.VectorSubcoreMesh(core_axis_name="c", subcore_axis_name="s",
                                    num_cores=NC, num_subcores=NS),
        scratch_types=(
            pltpu.VMEM((T,), jnp.int32),
            pltpu.VMEM((T,), jnp.float32),
            pltpu.VMEM((SLOTS_PAD,), jnp.int32),
            pltpu.VMEM((SPW,), jnp.float32),
            pltpu.VMEM((_HALF, DIM), jnp.float32),
            pltpu.VMEM((_HALF, DIM), jnp.float32),
            pltpu.SemaphoreType.DMA,
            pltpu.SemaphoreType.DMA,
            pltpu.SemaphoreType.DMA,
        ),
        compiler_params=pltpu.CompilerParams(needs_layout_passes=False),
    )


_EB = 2
_NSTEP = E // _EB + 1


def _ffn_body(disp_ref, w1_ref, b1_ref, w2_ref, b2_ref, swt_ref, out_ref):
    g = pl.program_id(0)

    @pl.when(g < E // _EB)
    def _():
        for j in range(_EB):
            x = disp_ref[pl.ds(j * CAP, CAP), :]
            h = lax.dot_general(x, w1_ref[j], (((1,), (0,)), ((), ())),
                                preferred_element_type=jnp.float32)
            h = jnp.maximum(h + b1_ref[pl.ds(g * _EB + j, 1), :], 0.0)
            y = lax.dot_general(h, w2_ref[j], (((1,), (0,)), ((), ())),
                                preferred_element_type=jnp.float32)
            y = y + b2_ref[pl.ds(g * _EB + j, 1), :]
            out_ref[pl.ds(j * CAP, CAP), :] = y * swt_ref[pl.ds(j * CAP, CAP), :]

    @pl.when(g == E // _EB)
    def _():
        out_ref[...] = jnp.zeros((_EB * CAP, DIM), jnp.float32)


def _ffn(disp, w1, b1, w2, b2, slot_weight):
    clamp3 = lambda g: (jnp.minimum(g, E // _EB - 1), 0, 0)
    clamp2 = lambda g: (jnp.minimum(g, E // _EB - 1), 0)
    return pl.pallas_call(
        _ffn_body,
        grid=(_NSTEP,),
        in_specs=[
            pl.BlockSpec((_EB * CAP, DIM), clamp2),
            pl.BlockSpec((_EB, DIM, DFF), clamp3),
            pl.BlockSpec((E, DFF), lambda g: (0, 0)),
            pl.BlockSpec((_EB, DFF, DIM), clamp3),
            pl.BlockSpec((E, DIM), lambda g: (0, 0)),
            pl.BlockSpec((_EB * CAP, 1), clamp2),
        ],
        out_specs=pl.BlockSpec((_EB * CAP, DIM), lambda g: (g, 0)),
        out_shape=jax.ShapeDtypeStruct((SLOTS + _EB * CAP, DIM), jnp.float32),
    )(disp, w1, b1, w2, b2, slot_weight)


_NCHUNK = 4
_CSZ = TPW // _NCHUNK


def _combine_body(eo_hbm, slot_hbm, out_hbm, idx_v,
                  rows0_v, rows1_v, rows2_v, rows3_v,
                  sem0, sem1, sem2, sem3):
    wid = lax.axis_index("s") * NC + lax.axis_index("c")
    base = wid * TPW
    pltpu.sync_copy(slot_hbm.at[pl.ds(base, TPW)], idx_v)
    rows = (rows0_v, rows1_v, rows2_v, rows3_v)
    sems = (sem0, sem1, sem2, sem3)
    gs = [pltpu.async_copy(eo_hbm.at[idx_v.at[pl.ds(i * _CSZ, _CSZ)]],
                           rows[i], sems[i])
          for i in range(_NCHUNK)]
    ws = []
    for i in range(_NCHUNK):
        gs[i].wait()
        ws.append(pltpu.async_copy(
            rows[i], out_hbm.at[pl.ds(base + i * _CSZ, _CSZ)], sems[i]))
    for w in ws:
        w.wait()


@functools.lru_cache(maxsize=None)
def _combine_kernel():
    return pl.kernel(
        _combine_body,
        out_type=jax.ShapeDtypeStruct((T, DIM), jnp.float32),
        mesh=plsc.VectorSubcoreMesh(core_axis_name="c", subcore_axis_name="s",
                                    num_cores=NC, num_subcores=NS),
        scratch_types=(
            pltpu.VMEM((TPW,), jnp.int32),
            pltpu.VMEM((_CSZ, DIM), jnp.float32),
            pltpu.VMEM((_CSZ, DIM), jnp.float32),
            pltpu.VMEM((_CSZ, DIM), jnp.float32),
            pltpu.VMEM((_CSZ, DIM), jnp.float32),
            pltpu.SemaphoreType.DMA,
            pltpu.SemaphoreType.DMA,
            pltpu.SemaphoreType.DMA,
            pltpu.SemaphoreType.DMA,
        ),
        compiler_params=pltpu.CompilerParams(needs_layout_passes=False),
    )


def kernel(inputs, gate_weight, w1, b1, w2, b2):
    tokens = inputs.reshape(T, DIM)
    token_slot, token_wt, laux = _route(tokens, gate_weight)
    disp, slot_weight = _dispatch_kernel()(tokens, token_slot, token_wt)
    eo = _ffn(disp, w1, b1, w2, b2, slot_weight.reshape(SLOTS, 1))
    ans = _combine_kernel()(eo, token_slot)
    return ans.reshape(inputs.shape), laux.reshape(())


# speedup vs baseline: 1.0367x; 1.0172x over previous
import functools

import jax
import jax.numpy as jnp
from jax import lax
from jax.experimental import pallas as pl
from jax.experimental.pallas import tpu as pltpu
from jax.experimental.pallas import tpu_sc as plsc

DIM = 768
E = 64
DFF = 768
T = 2048
CAP = 40
SLOTS = E * CAP
NC = 2
NS = 16
NW = NC * NS
SPW = SLOTS // NW
TPW = T // NW
SLOTS_PAD = 2576


def _route_body(tok_ref, gw_ref, slot_ref, wt_ref, laux_ref):
    x = tok_ref[...]
    g = gw_ref[...]
    logits = lax.dot_general(g, x, (((1,), (1,)), ((), ())),
                             preferred_element_type=jnp.float32)
    m = jnp.max(logits, axis=0, keepdims=True)
    ex = jnp.exp(logits - m)
    probs = ex / jnp.sum(ex, axis=0, keepdims=True)
    pmax = jnp.max(probs, axis=0, keepdims=True)
    iota_e = lax.broadcasted_iota(jnp.int32, (E, T), 0)
    cand = jnp.where(probs == pmax, iota_e, jnp.int32(E))
    top1 = jnp.min(cand, axis=0, keepdims=True)
    mask = (iota_e == top1).astype(jnp.float32)
    c = mask
    shift = 1
    while shift < T:
        c = c + jnp.concatenate(
            [jnp.zeros((E, shift), jnp.float32), c[:, :T - shift]], axis=1)
        shift *= 2
    rank = jnp.sum((c - 1.0) * mask, axis=0)
    kept = rank < float(CAP)
    wsel = jnp.sum(probs * mask, axis=0)
    slot = jnp.where(kept,
                     jnp.min(cand, axis=0) * CAP + rank.astype(jnp.int32),
                     jnp.int32(SLOTS))
    me = jnp.sum(probs, axis=1, keepdims=True) / float(T)
    ce = jnp.sum(mask, axis=1, keepdims=True) / float(T)
    laux_ref[...] = float(E) * jnp.sum(me * ce, axis=0, keepdims=True)
    slot_ref[...] = slot
    wt_ref[...] = wsel


def _route(tokens, gate_weight):
    return pl.pallas_call(
        _route_body,
        out_shape=(
            jax.ShapeDtypeStruct((T,), jnp.int32),
            jax.ShapeDtypeStruct((T,), jnp.float32),
            jax.ShapeDtypeStruct((1, 1), jnp.float32),
        ),
    )(tokens, gate_weight)


_UNROLL = 8
_HALF = SPW // 2


def _dispatch_body(tok_hbm, slot_hbm, wt_hbm, disp_hbm, swt_hbm,
                   slot_v, wt_v, s2t_v, sw_v, rows0_v, rows1_v,
                   sem0, sem1, sem2):
    wid = lax.axis_index("s") * NC + lax.axis_index("c")
    base = wid * SPW
    with jax.named_scope("disp_load_in"):
        lw = pltpu.async_copy(wt_hbm, wt_v, sem2)
        pltpu.sync_copy(slot_hbm, slot_v)

    for i in range(SPW // 16):
        s2t_v[pl.ds(base + i * 16, 16)] = (
            (lax.iota(jnp.int32, 16) + (base + i * 16)) & (T - 1))

    def scatter(i, _):
        for j in range(_UNROLL):
            k = i * _UNROLL + j
            idx = slot_v[pl.ds(k * 16, 16)]
            plsc.store_scatter(s2t_v, [idx],
                               lax.iota(jnp.int32, 16) + k * 16)
        return 0

    with jax.named_scope("disp_scatter"):
        lax.fori_loop(0, T // 16 // _UNROLL, scatter, 0)

    g0 = pltpu.async_copy(tok_hbm.at[s2t_v.at[pl.ds(base, _HALF)]],
                          rows0_v, sem0)
    g1 = pltpu.async_copy(tok_hbm.at[s2t_v.at[pl.ds(base + _HALF, _HALF)]],
                          rows1_v, sem1)
    with jax.named_scope("disp_sw"):
        lw.wait()
        for i in range(SPW // 16):
            idx = s2t_v[pl.ds(base + i * 16, 16)]
            sw_v[pl.ds(i * 16, 16)] = plsc.load_gather(wt_v, [idx])
        pltpu.sync_copy(sw_v, swt_hbm.at[pl.ds(base, SPW)])
    with jax.named_scope("disp_rows"):
        g0.wait()
        w0 = pltpu.async_copy(rows0_v, disp_hbm.at[pl.ds(base, _HALF)], sem0)
        g1.wait()
        w1 = pltpu.async_copy(rows1_v,
                              disp_hbm.at[pl.ds(base + _HALF, _HALF)], sem1)
        w0.wait()
        w1.wait()


@functools.lru_cache(maxsize=None)
def _dispatch_kernel():
    return pl.kernel(
        _dispatch_body,
        out_type=(
            jax.ShapeDtypeStruct((SLOTS, DIM), jnp.float32),
            jax.ShapeDtypeStruct((SLOTS,), jnp.float32),
        ),
        mesh=plsc.VectorSubcoreMesh(core_axis_name="c", subcore_axis_name="s",
                                    num_cores=NC, num_subcores=NS),
        scratch_types=(
            pltpu.VMEM((T,), jnp.int32),
            pltpu.VMEM((T,), jnp.float32),
            pltpu.VMEM((SLOTS_PAD,), jnp.int32),
            pltpu.VMEM((SPW,), jnp.float32),
            pltpu.VMEM((_HALF, DIM), jnp.float32),
            pltpu.VMEM((_HALF, DIM), jnp.float32),
            pltpu.SemaphoreType.DMA,
            pltpu.SemaphoreType.DMA,
            pltpu.SemaphoreType.DMA,
        ),
        compiler_params=pltpu.CompilerParams(needs_layout_passes=False),
    )


_EB = 2
_NSTEP = E // _EB + 1


def _ffn_body(disp_ref, w1_ref, b1_ref, w2_ref, b2_ref, swt_ref, out_ref):
    g = pl.program_id(0)

    @pl.when(g < E // _EB)
    def _():
        for j in range(_EB):
            x = disp_ref[pl.ds(j * CAP, CAP), :]
            h = lax.dot_general(x, w1_ref[j], (((1,), (0,)), ((), ())),
                                preferred_element_type=jnp.float32)
            h = jnp.maximum(h + b1_ref[pl.ds(g * _EB + j, 1), :], 0.0)
            y = lax.dot_general(h, w2_ref[j], (((1,), (0,)), ((), ())),
                                preferred_element_type=jnp.float32)
            y = y + b2_ref[pl.ds(g * _EB + j, 1), :]
            sbase = (g * _EB + j) * CAP
            for r in range(CAP):
                out_ref[pl.ds(j * CAP + r, 1), :] = (
                    y[r:r + 1, :] * swt_ref[sbase + r])

    @pl.when(g == E // _EB)
    def _():
        out_ref[...] = jnp.zeros((_EB * CAP, DIM), jnp.float32)


def _ffn(disp, w1, b1, w2, b2, slot_weight):
    clamp3 = lambda g: (jnp.minimum(g, E // _EB - 1), 0, 0)
    clamp2 = lambda g: (jnp.minimum(g, E // _EB - 1), 0)
    return pl.pallas_call(
        _ffn_body,
        grid=(_NSTEP,),
        in_specs=[
            pl.BlockSpec((_EB * CAP, DIM), clamp2),
            pl.BlockSpec((_EB, DIM, DFF), clamp3),
            pl.BlockSpec((E, DFF), lambda g: (0, 0)),
            pl.BlockSpec((_EB, DFF, DIM), clamp3),
            pl.BlockSpec((E, DIM), lambda g: (0, 0)),
            pl.BlockSpec(memory_space=pltpu.SMEM),
        ],
        out_specs=pl.BlockSpec((_EB * CAP, DIM), lambda g: (g, 0)),
        out_shape=jax.ShapeDtypeStruct((SLOTS + _EB * CAP, DIM), jnp.float32),
    )(disp, w1, b1, w2, b2, slot_weight)


_NCHUNK = 4
_CSZ = TPW // _NCHUNK


def _combine_body(eo_hbm, slot_hbm, out_hbm, idx_v,
                  rows0_v, rows1_v, rows2_v, rows3_v,
                  sem0, sem1, sem2, sem3):
    wid = lax.axis_index("s") * NC + lax.axis_index("c")
    base = wid * TPW
    pltpu.sync_copy(slot_hbm.at[pl.ds(base, TPW)], idx_v)
    rows = (rows0_v, rows1_v, rows2_v, rows3_v)
    sems = (sem0, sem1, sem2, sem3)
    gs = [pltpu.async_copy(eo_hbm.at[idx_v.at[pl.ds(i * _CSZ, _CSZ)]],
                           rows[i], sems[i])
          for i in range(_NCHUNK)]
    ws = []
    for i in range(_NCHUNK):
        gs[i].wait()
        ws.append(pltpu.async_copy(
            rows[i], out_hbm.at[pl.ds(base + i * _CSZ, _CSZ)], sems[i]))
    for w in ws:
        w.wait()


@functools.lru_cache(maxsize=None)
def _combine_kernel():
    return pl.kernel(
        _combine_body,
        out_type=jax.ShapeDtypeStruct((T, DIM), jnp.float32),
        mesh=plsc.VectorSubcoreMesh(core_axis_name="c", subcore_axis_name="s",
                                    num_cores=NC, num_subcores=NS),
        scratch_types=(
            pltpu.VMEM((TPW,), jnp.int32),
            pltpu.VMEM((_CSZ, DIM), jnp.float32),
            pltpu.VMEM((_CSZ, DIM), jnp.float32),
            pltpu.VMEM((_CSZ, DIM), jnp.float32),
            pltpu.VMEM((_CSZ, DIM), jnp.float32),
            pltpu.SemaphoreType.DMA,
            pltpu.SemaphoreType.DMA,
            pltpu.SemaphoreType.DMA,
            pltpu.SemaphoreType.DMA,
        ),
        compiler_params=pltpu.CompilerParams(needs_layout_passes=False),
    )


def kernel(inputs, gate_weight, w1, b1, w2, b2):
    tokens = inputs.reshape(T, DIM)
    token_slot, token_wt, laux = _route(tokens, gate_weight)
    disp, slot_weight = _dispatch_kernel()(tokens, token_slot, token_wt)
    eo = _ffn(disp, w1, b1, w2, b2, slot_weight)
    ans = _combine_kernel()(eo, token_slot)
    return ans.reshape(inputs.shape), laux.reshape(())
